# baseline (device time: 24434 ns/iter reference)
import jax
import jax.numpy as jnp
from jax import lax
from jax.experimental import pallas as pl
from jax.experimental.pallas import tpu as pltpu

MESH = pl.DeviceIdType.MESH


def kernel(x):
    _, m, n = x.shape
    half = n // 2
    qw = half // 4
    mh = m // 2

    omx = lax.axis_index("x")
    omy = lax.axis_index("y")
    omz = lax.axis_index("z")
    oq_own = 2 * omx + omz
    oq_d = 2 * (1 - omx) + (1 - omz)
    peer_base = (1 - omy) * half
    x2 = x[0]
    xb = jnp.concatenate(
        [
            lax.dynamic_slice_in_dim(x2, peer_base + oq_own * qw, qw, axis=1),
            lax.dynamic_slice_in_dim(x2, peer_base + oq_d * qw, qw, axis=1),
            lax.dynamic_slice_in_dim(x2, omy * half, half, axis=1),
        ],
        axis=1,
    ).astype(jnp.bfloat16)

    def body(x_ref, out_ref, yrecv, x2recv, z2recv, send_sems, recv_sems):
        mx = lax.axis_index("x")
        my = lax.axis_index("y")
        mz = lax.axis_index("z")
        peer_y = (mx, 1 - my, mz)
        nbr_x = (1 - mx, my, mz)
        nbr_z = (mx, my, 1 - mz)

        q_own = 2 * mx + mz
        q_x = 2 * (1 - mx) + mz
        q_z = 2 * mx + (1 - mz)
        q_d = 2 * (1 - mx) + (1 - mz)

        barrier_sem = pltpu.get_barrier_semaphore()
        for nbr in (peer_y, nbr_x, nbr_z):
            pl.semaphore_signal(barrier_sem, inc=1, device_id=nbr,
                                device_id_type=MESH)
        pl.semaphore_wait(barrier_sem, 3)

        def exchange(src, dst, sem_idx, dev):
            r = pltpu.make_async_remote_copy(
                src_ref=src, dst_ref=dst,
                send_sem=send_sems.at[sem_idx],
                recv_sem=recv_sems.at[sem_idx],
                device_id=dev, device_id_type=MESH,
            )
            r.start()
            return r

        def add_quarter_half(q, h, contrib):
            rows = pl.ds(h * mh, mh)
            mine = x_ref[rows, pl.ds(2 * qw + q * qw, qw)]
            out_ref[rows, pl.ds(q * qw, qw)] = mine + contrib

        ry = []
        for qi in (0, 1):
            for h in (0, 1):
                src = x_ref.at[pl.ds(h * mh, mh), pl.ds(qi * qw, qw)]
                ry.append(
                    exchange(src, yrecv.at[qi, h], 2 * qi + h, peer_y)
                )

        rx, rz = [], []
        for h in (0, 1):
            ry[h].wait_recv()
            rx.append(exchange(yrecv.at[0, h], x2recv.at[h], 4 + h, nbr_x))
            rz.append(exchange(yrecv.at[0, h], z2recv.at[h], 6 + h, nbr_z))
            add_quarter_half(q_own, h, yrecv[0, h])

        for h in (0, 1):
            ry[2 + h].wait_recv()
            add_quarter_half(q_d, h, yrecv[1, h])

        for h in (0, 1):
            rx[h].wait_recv()
            add_quarter_half(q_x, h, x2recv[h])
        for h in (0, 1):
            rz[h].wait_recv()
            add_quarter_half(q_z, h, z2recv[h])

        for r in ry + rx + rz:
            r.wait_send()

    return pl.pallas_call(
        body,
        out_shape=jax.ShapeDtypeStruct((m, half), jnp.bfloat16),
        in_specs=[pl.BlockSpec(memory_space=pltpu.VMEM)],
        out_specs=pl.BlockSpec(memory_space=pltpu.VMEM),
        scratch_shapes=[
            pltpu.VMEM((2, 2, mh, qw), jnp.bfloat16),
            pltpu.VMEM((2, mh, qw), jnp.bfloat16),
            pltpu.VMEM((2, mh, qw), jnp.bfloat16),
            pltpu.SemaphoreType.DMA((8,)),
            pltpu.SemaphoreType.DMA((8,)),
        ],
        compiler_params=pltpu.CompilerParams(collective_id=0),
    )(xb)


# device time: 22279 ns/iter; 1.0967x vs baseline; 1.0967x over previous
import jax
import jax.numpy as jnp
from jax import lax
from jax.experimental import pallas as pl
from jax.experimental.pallas import tpu as pltpu

MESH = pl.DeviceIdType.MESH


def kernel(x):
    _, m, n = x.shape
    half = n // 2
    qw = half // 4
    mh = m // 2

    xb = x[0].astype(jnp.bfloat16)

    mq = m // 4

    def body(x_ref, out_ref, yrecv_own, yrecv_diag, x2recv, z2recv,
             send_sems, recv_sems):
        mx = lax.axis_index("x")
        my = lax.axis_index("y")
        mz = lax.axis_index("z")
        peer_y = (mx, 1 - my, mz)
        nbr_x = (1 - mx, my, mz)
        nbr_z = (mx, my, 1 - mz)

        my_base = my * half
        peer_base = (1 - my) * half
        q_own = 2 * mx + mz
        q_x = 2 * (1 - mx) + mz
        q_z = 2 * mx + (1 - mz)
        q_d = 2 * (1 - mx) + (1 - mz)

        barrier_sem = pltpu.get_barrier_semaphore()
        for nbr in (peer_y, nbr_x, nbr_z):
            pl.semaphore_signal(barrier_sem, inc=1, device_id=nbr,
                                device_id_type=MESH)
        pl.semaphore_wait(barrier_sem, 3)

        def exchange(src, dst, sem_idx, dev):
            r = pltpu.make_async_remote_copy(
                src_ref=src, dst_ref=dst,
                send_sem=send_sems.at[sem_idx],
                recv_sem=recv_sems.at[sem_idx],
                device_id=dev, device_id_type=MESH,
            )
            r.start()
            return r

        def add_quarter_rows(q, k, nk, contrib):
            rq = m // nk
            rows = pl.ds(k * rq, rq)
            mine = x_ref[rows, pl.ds(my_base + q * qw, qw)]
            out_ref[rows, pl.ds(q * qw, qw)] = mine + contrib

        ry_own = []
        for k in range(4):
            src = x_ref.at[
                pl.ds(k * mq, mq), pl.ds(peer_base + q_own * qw, qw)
            ]
            ry_own.append(exchange(src, yrecv_own.at[k], k, peer_y))
        ry_diag = []
        for h in (0, 1):
            src = x_ref.at[
                pl.ds(h * mh, mh), pl.ds(peer_base + q_d * qw, qw)
            ]
            ry_diag.append(exchange(src, yrecv_diag.at[h], 4 + h, peer_y))

        rx, rz = [], []
        for k in range(4):
            ry_own[k].wait_recv()
            rx.append(exchange(yrecv_own.at[k], x2recv.at[k], 6 + k, nbr_x))
            rz.append(exchange(yrecv_own.at[k], z2recv.at[k], 10 + k, nbr_z))
            add_quarter_rows(q_own, k, 4, yrecv_own[k])

        for h in (0, 1):
            ry_diag[h].wait_recv()
            add_quarter_rows(q_d, h, 2, yrecv_diag[h])

        for k in range(4):
            rx[k].wait_recv()
            add_quarter_rows(q_x, k, 4, x2recv[k])
        for k in range(4):
            rz[k].wait_recv()
            add_quarter_rows(q_z, k, 4, z2recv[k])

        for r in ry_own + ry_diag + rx + rz:
            r.wait_send()

    return pl.pallas_call(
        body,
        out_shape=jax.ShapeDtypeStruct((m, half), jnp.bfloat16),
        in_specs=[pl.BlockSpec(memory_space=pltpu.VMEM)],
        out_specs=pl.BlockSpec(memory_space=pltpu.VMEM),
        scratch_shapes=[
            pltpu.VMEM((4, mq, qw), jnp.bfloat16),
            pltpu.VMEM((2, mh, qw), jnp.bfloat16),
            pltpu.VMEM((4, mq, qw), jnp.bfloat16),
            pltpu.VMEM((4, mq, qw), jnp.bfloat16),
            pltpu.SemaphoreType.DMA((14,)),
            pltpu.SemaphoreType.DMA((14,)),
        ],
        compiler_params=pltpu.CompilerParams(collective_id=0),
    )(xb)


# device time: 21831 ns/iter; 1.1192x vs baseline; 1.0205x over previous
import jax
import jax.numpy as jnp
from jax import lax
from jax.experimental import pallas as pl
from jax.experimental.pallas import tpu as pltpu

MESH = pl.DeviceIdType.MESH


def kernel(x):
    _, m, n = x.shape
    half = n // 2
    qw = half // 4
    mh = m // 2

    def body(x_ref, out_ref, ysend, yrecv, x2recv, z2recv,
             send_sems, recv_sems):
        mx = lax.axis_index("x")
        my = lax.axis_index("y")
        mz = lax.axis_index("z")
        peer_y = (mx, 1 - my, mz)
        nbr_x = (1 - mx, my, mz)
        nbr_z = (mx, my, 1 - mz)

        my_base = my * half
        peer_base = (1 - my) * half
        q_own = 2 * mx + mz
        q_x = 2 * (1 - mx) + mz
        q_z = 2 * mx + (1 - mz)
        q_d = 2 * (1 - mx) + (1 - mz)

        barrier_sem = pltpu.get_barrier_semaphore()
        for nbr in (peer_y, nbr_x, nbr_z):
            pl.semaphore_signal(barrier_sem, inc=1, device_id=nbr,
                                device_id_type=MESH)
        pl.semaphore_wait(barrier_sem, 3)

        def exchange(src, dst, sem_idx, dev):
            r = pltpu.make_async_remote_copy(
                src_ref=src, dst_ref=dst,
                send_sem=send_sems.at[sem_idx],
                recv_sem=recv_sems.at[sem_idx],
                device_id=dev, device_id_type=MESH,
            )
            r.start()
            return r

        def add_quarter_half(q, h, contrib):
            rows = pl.ds(h * mh, mh)
            mine = x_ref[0, rows, pl.ds(my_base + q * qw, qw)]
            out_ref[rows, pl.ds(q * qw, qw)] = (
                mine + contrib.astype(jnp.float32)
            ).astype(jnp.bfloat16)

        ry = []
        for qi, col_q in ((0, q_own), (1, q_d)):
            for h in (0, 1):
                ysend[qi, h] = x_ref[
                    0, pl.ds(h * mh, mh), pl.ds(peer_base + col_q * qw, qw)
                ].astype(jnp.bfloat16)
                ry.append(
                    exchange(ysend.at[qi, h], yrecv.at[qi, h],
                             2 * qi + h, peer_y)
                )

        rx, rz = [], []
        for h in (0, 1):
            ry[h].wait_recv()
            rx.append(exchange(yrecv.at[0, h], x2recv.at[h], 4 + h, nbr_x))
            rz.append(exchange(yrecv.at[0, h], z2recv.at[h], 6 + h, nbr_z))
            add_quarter_half(q_own, h, yrecv[0, h])

        for h in (0, 1):
            ry[2 + h].wait_recv()
            add_quarter_half(q_d, h, yrecv[1, h])

        for h in (0, 1):
            rx[h].wait_recv()
            add_quarter_half(q_x, h, x2recv[h])
        for h in (0, 1):
            rz[h].wait_recv()
            add_quarter_half(q_z, h, z2recv[h])

        for r in ry + rx + rz:
            r.wait_send()

    return pl.pallas_call(
        body,
        out_shape=jax.ShapeDtypeStruct((m, half), jnp.bfloat16),
        in_specs=[pl.BlockSpec(memory_space=pltpu.VMEM)],
        out_specs=pl.BlockSpec(memory_space=pltpu.VMEM),
        scratch_shapes=[
            pltpu.VMEM((2, 2, mh, qw), jnp.bfloat16),
            pltpu.VMEM((2, 2, mh, qw), jnp.bfloat16),
            pltpu.VMEM((2, mh, qw), jnp.bfloat16),
            pltpu.VMEM((2, mh, qw), jnp.bfloat16),
            pltpu.SemaphoreType.DMA((8,)),
            pltpu.SemaphoreType.DMA((8,)),
        ],
        compiler_params=pltpu.CompilerParams(collective_id=0),
    )(x)
